# baseline (speedup 1.0000x reference)
import functools

import jax
import jax.numpy as jnp
import numpy as np
from jax.experimental import pallas as pl
from jax.experimental.pallas import tpu as pltpu


def _sel(ndj, nw, nj):
    t = np.zeros((ndj, nw, nj), np.float32)
    for dj in range(ndj):
        for j in range(nj):
            w = 2 * j - 1 + dj
            if 0 <= w < nw:
                t[dj, w, j] = 1.0
    return t


_T1 = _sel(3, 32, 16)
_T2 = _sel(3, 16, 8)


def _fused_kernel(bt, x_ref, a_ref, b_ref, b1_ref, b2_ref, fc_ref, fcb_ref,
                  lw_ref, lb_ref, rg_ref, rgb_ref, out_last_ref, out_reg_ref):
    f32 = jnp.float32
    bf16 = jnp.bfloat16
    dot = functools.partial(jnp.dot, preferred_element_type=f32)

    xb = x_ref[...].astype(bf16)
    xp = []
    for p in range(4):
        rows = []
        for i2 in range(8):
            h = 4 * i2 + p
            rows.append(jnp.concatenate(
                [xb[:, c * 1024 + h * 32: c * 1024 + h * 32 + 32]
                 for c in range(4)], axis=1))
        xp.append(jnp.concatenate(rows, axis=0))
    x0, x1, x2, x3 = xp

    zx = jnp.zeros((bt, 128), bf16)
    x3s = jnp.concatenate([zx, x3[: 7 * bt]], axis=0)
    xcat = jnp.concatenate([
        jnp.concatenate([x3s, x0, x1], axis=1),
        jnp.concatenate([x1, x2, x3], axis=1)], axis=0)
    b1 = jnp.tile(b1_ref[...], (1, 16))
    h_eo = jnp.maximum(dot(xcat, a_ref[...]) + b1, 0.0).astype(bf16)
    h_e, h_o = h_eo[: 8 * bt], h_eo[8 * bt:]

    zh = jnp.zeros((bt, 256), bf16)
    h_os = jnp.concatenate([zh, h_o[: 7 * bt]], axis=0)
    hcat = jnp.concatenate([h_os, h_e, h_o], axis=1)
    b2 = jnp.tile(b2_ref[...], (1, 8))
    out2 = jnp.maximum(dot(hcat, b_ref[...]) + b2, 0.0).astype(bf16)

    flat = jnp.concatenate(
        [out2[i2 * bt:(i2 + 1) * bt] for i2 in range(8)], axis=1)
    h = jnp.maximum(dot(flat, fc_ref[...]) + fcb_ref[...], 0.0).astype(bf16)
    out_last_ref[...] = dot(h, lw_ref[...]) + lb_ref[...]
    out_reg_ref[...] = dot(flat, rg_ref[...]) + rgb_ref[...]


def kernel(x_flat, conv0_w, conv0_b, conv1_w, conv1_b, fc0_w, fc0_b,
           last_w, last_b, reg_w, reg_b):
    f32 = jnp.float32
    bf16 = jnp.bfloat16
    B = x_flat.shape[0]
    bt = 256 if B % 256 == 0 else B

    amat = jnp.einsum("dwj,ocid->icwjo", _T1, conv0_w).reshape(384, 256)
    bmat = jnp.einsum("dab,ocid->iacbo", _T2, conv1_w).reshape(768, 256)

    fcr = fc0_w.reshape(32, 8, 8, 256).transpose(1, 2, 0, 3).reshape(2048, 256)
    rgr = reg_w.reshape(32, 8, 8, 64).transpose(1, 2, 0, 3).reshape(2048, 64)

    full = lambda a: pl.BlockSpec(a.shape, lambda i: (0,) * a.ndim)
    weights = [amat.astype(bf16), bmat.astype(bf16),
               conv0_b.reshape(1, 16), conv1_b.reshape(1, 32),
               fcr.astype(bf16), fc0_b.reshape(1, 256),
               last_w.astype(bf16), last_b.reshape(1, 128),
               rgr.astype(bf16), reg_b.reshape(1, 64)]

    out_last, out_reg = pl.pallas_call(
        functools.partial(_fused_kernel, bt),
        out_shape=(jax.ShapeDtypeStruct((B, 128), f32),
                   jax.ShapeDtypeStruct((B, 64), f32)),
        grid=(B // bt,),
        in_specs=[pl.BlockSpec((bt, 4096), lambda i: (i, 0))]
        + [full(w) for w in weights],
        out_specs=[pl.BlockSpec((bt, 128), lambda i: (i, 0)),
                   pl.BlockSpec((bt, 64), lambda i: (i, 0))],
        compiler_params=pltpu.CompilerParams(
            dimension_semantics=("parallel",)),
    )(x_flat, *weights)
    return out_last, out_reg


# speedup vs baseline: 1.0102x; 1.0102x over previous
import functools

import jax
import jax.numpy as jnp
import numpy as np
from jax.experimental import pallas as pl
from jax.experimental.pallas import tpu as pltpu


def _sel(ndj, nw, nj):
    t = np.zeros((ndj, nw, nj), np.float32)
    for dj in range(ndj):
        for j in range(nj):
            w = 2 * j - 1 + dj
            if 0 <= w < nw:
                t[dj, w, j] = 1.0
    return t


_T1 = _sel(3, 32, 16)
_T2 = _sel(3, 16, 8)


def _fused_kernel(bt, x_ref, a_ref, b_ref, b1_ref, b2_ref, fc_ref, fcb_ref,
                  lw_ref, lb_ref, rgb_ref, out_last_ref, out_reg_ref):
    f32 = jnp.float32
    bf16 = jnp.bfloat16
    dot = functools.partial(jnp.dot, preferred_element_type=f32)

    xb = x_ref[...].astype(bf16)
    xp = []
    for p in range(4):
        rows = []
        for i2 in range(8):
            h = 4 * i2 + p
            rows.append(jnp.concatenate(
                [xb[:, c * 1024 + h * 32: c * 1024 + h * 32 + 32]
                 for c in range(4)], axis=1))
        xp.append(jnp.concatenate(rows, axis=0))
    x0, x1, x2, x3 = xp

    zx = jnp.zeros((bt, 128), bf16)
    x3s = jnp.concatenate([zx, x3[: 7 * bt]], axis=0)
    xcat = jnp.concatenate([
        jnp.concatenate([x3s, x0, x1], axis=1),
        jnp.concatenate([x1, x2, x3], axis=1)], axis=0)
    b1 = jnp.tile(b1_ref[...], (1, 16))
    h_eo = jnp.maximum(dot(xcat, a_ref[...]) + b1, 0.0).astype(bf16)
    h_e, h_o = h_eo[: 8 * bt], h_eo[8 * bt:]

    zh = jnp.zeros((bt, 256), bf16)
    h_os = jnp.concatenate([zh, h_o[: 7 * bt]], axis=0)
    hcat = jnp.concatenate([h_os, h_e, h_o], axis=1)
    b2 = jnp.tile(b2_ref[...], (1, 8))
    out2 = jnp.maximum(dot(hcat, b_ref[...]) + b2, 0.0).astype(bf16)

    flat = jnp.concatenate(
        [out2[i2 * bt:(i2 + 1) * bt] for i2 in range(8)], axis=1)
    hr = dot(flat, fc_ref[...])
    h = jnp.maximum(hr[:, :256] + fcb_ref[...], 0.0).astype(bf16)
    out_last_ref[...] = dot(h, lw_ref[...]) + lb_ref[...]
    out_reg_ref[...] = hr[:, 256:] + rgb_ref[...]


def kernel(x_flat, conv0_w, conv0_b, conv1_w, conv1_b, fc0_w, fc0_b,
           last_w, last_b, reg_w, reg_b):
    f32 = jnp.float32
    bf16 = jnp.bfloat16
    B = x_flat.shape[0]
    bt = 512 if B % 512 == 0 else B

    amat = jnp.einsum("dwj,ocid->icwjo", _T1, conv0_w).reshape(384, 256)
    bmat = jnp.einsum("dab,ocid->iacbo", _T2, conv1_w).reshape(768, 256)

    fcr = fc0_w.reshape(32, 8, 8, 256).transpose(1, 2, 0, 3).reshape(2048, 256)
    rgr = reg_w.reshape(32, 8, 8, 64).transpose(1, 2, 0, 3).reshape(2048, 64)

    fcrg = jnp.concatenate([fcr, rgr], axis=1)
    full = lambda a: pl.BlockSpec(a.shape, lambda i: (0,) * a.ndim)
    weights = [amat.astype(bf16), bmat.astype(bf16),
               conv0_b.reshape(1, 16), conv1_b.reshape(1, 32),
               fcrg.astype(bf16), fc0_b.reshape(1, 256),
               last_w.astype(bf16), last_b.reshape(1, 128),
               reg_b.reshape(1, 64)]

    out_last, out_reg = pl.pallas_call(
        functools.partial(_fused_kernel, bt),
        out_shape=(jax.ShapeDtypeStruct((B, 128), f32),
                   jax.ShapeDtypeStruct((B, 64), f32)),
        grid=(B // bt,),
        in_specs=[pl.BlockSpec((bt, 4096), lambda i: (i, 0))]
        + [full(w) for w in weights],
        out_specs=[pl.BlockSpec((bt, 128), lambda i: (i, 0)),
                   pl.BlockSpec((bt, 64), lambda i: (i, 0))],
        compiler_params=pltpu.CompilerParams(
            dimension_semantics=("parallel",)),
    )(x_flat, *weights)
    return out_last, out_reg
